# baseline (speedup 1.0000x reference)
import functools

import jax
import jax.numpy as jnp
from jax import lax
from jax.experimental import pallas as pl
from jax.experimental.pallas import tpu as pltpu
from jax.experimental.pallas import tpu_sc as plsc

B, F = 4096, 26
V, D = 100000, 64
FP = 32
L = 16
NC, NS = 2, 16
NW = NC * NS
ROWS_W = B // NW
NBLK = ROWS_W // L
IDX_BLK = F * L
NCH = 4
CH = IDX_BLK // NCH
CHF = FP * L // NCH
VS_BLK = IDX_BLK // 8
DCH = D // L


def _sc_body(idx_hbm, idxfo_hbm, vsplat_hbm, fo_hbm, so_hbm,
             pooled_hbm, sumsq_hbm, fow_hbm,
             idx_v, idxfo_v, vs_v, emb_v, fow_v, pooled_v, sumsq_v,
             sem_e, sem_f, sem_v):
    wid = lax.axis_index("s") * NC + lax.axis_index("c")
    pltpu.sync_copy(idx_hbm.at[pl.ds(wid * (NBLK * NCH), NBLK * NCH)], idx_v)
    pltpu.sync_copy(idxfo_hbm.at[pl.ds(wid * (NBLK * NCH), NBLK * NCH)],
                    idxfo_v)
    pltpu.sync_copy(
        vsplat_hbm.at[pl.ds(wid * (NBLK * IDX_BLK), NBLK * IDX_BLK)], vs_v)

    def block_body(blk, carry):
        for c in range(NCH):
            pltpu.async_copy(so_hbm.at[idx_v.at[blk * NCH + c]],
                             emb_v.at[pl.ds(c * CH, CH)], sem_e)
        for c in range(NCH):
            pltpu.async_copy(fo_hbm.at[idxfo_v.at[blk * NCH + c]],
                             fow_v.at[pl.ds(blk * NCH * CHF + c * CHF, CHF)],
                             sem_f)
        for c in range(NCH):
            pltpu.make_async_copy(so_hbm.at[idx_v.at[blk * NCH + c]],
                                  emb_v.at[pl.ds(c * CH, CH)], sem_e).wait()
        for c in range(NCH):
            pltpu.make_async_copy(
                fo_hbm.at[idxfo_v.at[blk * NCH + c]],
                fow_v.at[pl.ds(blk * NCH * CHF + c * CHF, CHF)],
                sem_f).wait()

        def row_body(r, rc):
            row = blk * L + r
            accs = [jnp.zeros((L,), jnp.float32) for _ in range(2 * DCH)]
            base = (blk * L + r) * F
            for f in range(F):
                jl = r * F + f
                vv = vs_v[base + f, :]
                for c in range(DCH):
                    t = emb_v[jl, pl.ds(c * L, L)] * vv
                    accs[c] = accs[c] + t
                    accs[DCH + c] = accs[DCH + c] + t * t
            for c in range(DCH):
                pooled_v[row, pl.ds(c * L, L)] = accs[c]
                sumsq_v[row, pl.ds(c * L, L)] = accs[DCH + c]
            return rc

        lax.fori_loop(0, L, row_body, 0)
        return carry

    lax.fori_loop(0, NBLK, block_body, 0)

    pltpu.sync_copy(pooled_v, pooled_hbm.at[pl.ds(wid * ROWS_W, ROWS_W)])
    pltpu.sync_copy(sumsq_v, sumsq_hbm.at[pl.ds(wid * ROWS_W, ROWS_W)])
    pltpu.sync_copy(fow_v, fow_hbm.at[pl.ds(wid * (ROWS_W * FP), ROWS_W * FP)])


@functools.cache
def _get_sc_call():
    return pl.kernel(
        _sc_body,
        out_type=(
            jax.ShapeDtypeStruct((B, D), jnp.float32),
            jax.ShapeDtypeStruct((B, D), jnp.float32),
            jax.ShapeDtypeStruct((B * FP,), jnp.float32),
        ),
        mesh=plsc.VectorSubcoreMesh(core_axis_name="c", subcore_axis_name="s"),
        compiler_params=pltpu.CompilerParams(use_tc_tiling_on_sc=False),
        scratch_types=(
            pltpu.VMEM((NBLK * NCH, CH), jnp.int32),
            pltpu.VMEM((NBLK * NCH, CHF), jnp.int32),
            pltpu.VMEM((NBLK * IDX_BLK, L), jnp.float32),
            pltpu.VMEM((IDX_BLK, D), jnp.float32),
            pltpu.VMEM((ROWS_W * FP,), jnp.float32),
            pltpu.VMEM((ROWS_W, D), jnp.float32),
            pltpu.VMEM((ROWS_W, D), jnp.float32),
            pltpu.SemaphoreType.DMA,
            pltpu.SemaphoreType.DMA,
            pltpu.SemaphoreType.DMA,
        ),
    )


def _tc_body(pooled_ref, sumsq_ref, fow_ref, fv_ref,
             W0_ref, b0_ref, g0_ref, be0_ref,
             W1_ref, b1_ref, g1_ref, be1_ref,
             W2_ref, b2_ref, g2_ref, be2_ref,
             Wo_ref, bo_ref, out_ref):
    p = pooled_ref[:]
    second = 0.5 * jnp.sum(p * p - sumsq_ref[:], axis=1)
    first = jnp.sum(fow_ref[:, :F] * fv_ref[:], axis=1)
    x = p
    for W_ref, b_ref, g_ref, be_ref in (
            (W0_ref, b0_ref, g0_ref, be0_ref),
            (W1_ref, b1_ref, g1_ref, be1_ref),
            (W2_ref, b2_ref, g2_ref, be2_ref)):
        x = lax.dot_general(x, W_ref[:], (((1,), (1,)), ((), ())),
                            preferred_element_type=jnp.float32) + b_ref[:]
        x = jnp.maximum(x, 0.0)
        mean = jnp.mean(x, axis=0, keepdims=True)
        var = jnp.mean((x - mean) ** 2, axis=0, keepdims=True)
        x = g_ref[:] * (x - mean) * lax.rsqrt(var + 1e-5) + be_ref[:]
    deep = lax.dot_general(x, Wo_ref[:], (((1,), (1,)), ((), ())),
                           preferred_element_type=jnp.float32)[:, 0]
    logit = first + second + deep + bo_ref[0]
    out_ref[:] = 1.0 / (1.0 + jnp.exp(-logit))


def kernel(feature_ids, feature_values, first_order_table, second_order_table,
           W0, b0, gamma0, beta0, W1, b1, gamma1, beta1, W2, b2, gamma2, beta2,
           W_out, b_out):
    idx = feature_ids.reshape(NW * NBLK * NCH, CH)
    idxfo = jnp.pad(feature_ids, ((0, 0), (0, FP - F))).reshape(
        NW * NBLK * NCH, CHF)
    vals_flat = feature_values.reshape(-1)
    vsplat = jnp.broadcast_to(vals_flat[:, None], (B * F, L))
    fo_flat = first_order_table.reshape(V)

    pooled, sumsq, fow = _get_sc_call()(idx, idxfo, vsplat, fo_flat,
                                        second_order_table)

    return pl.pallas_call(
        _tc_body,
        out_shape=jax.ShapeDtypeStruct((B,), jnp.float32),
    )(pooled, sumsq, fow.reshape(B, FP), feature_values,
      W0, b0, gamma0, beta0, W1, b1, gamma1, beta1,
      W2, b2, gamma2, beta2, W_out, b_out)


# speedup vs baseline: 2.1714x; 2.1714x over previous
import functools

import jax
import jax.numpy as jnp
from jax import lax
from jax.experimental import pallas as pl
from jax.experimental.pallas import tpu as pltpu
from jax.experimental.pallas import tpu_sc as plsc

B, F = 4096, 26
V, D = 100000, 64
FP = 32
L = 16
NC, NS = 2, 16
NW = NC * NS
ROWS_W = B // NW
NBLK = ROWS_W // L
DCH = D // L


def _sc_body(ids_hbm, vs_hbm, fo_hbm, so_hbm,
             pooled_hbm, sumsq_hbm, fow_hbm,
             ids_v, vs_v, emb_v, fow_v, pooled_v, sumsq_v,
             sem_e, sem_f, sem_v):
    wid = lax.axis_index("s") * NC + lax.axis_index("c")
    row0 = wid * ROWS_W
    pltpu.sync_copy(ids_hbm.at[pl.ds(row0, ROWS_W)], ids_v)
    pltpu.sync_copy(vs_hbm.at[pl.ds(row0 * (FP // 8), ROWS_W * (FP // 8))],
                    vs_v)

    def block_body(blk, carry):
        for rl in range(L):
            wr = blk * L + rl
            pltpu.async_copy(so_hbm.at[ids_v.at[wr]], emb_v.at[rl], sem_e)
        for rl in range(L):
            wr = blk * L + rl
            pltpu.async_copy(fo_hbm.at[ids_v.at[wr]],
                             fow_v.at[pl.ds(wr * FP, F)], sem_f)
        for rl in range(L):
            wr = blk * L + rl
            pltpu.make_async_copy(so_hbm.at[ids_v.at[wr]], emb_v.at[rl],
                                  sem_e).wait()
        for rl in range(L):
            wr = blk * L + rl
            pltpu.make_async_copy(fo_hbm.at[ids_v.at[wr]],
                                  fow_v.at[pl.ds(wr * FP, F)], sem_f).wait()
        def row_body(rl, rc):
            row = blk * L + rl
            accs = [jnp.zeros((L,), jnp.float32) for _ in range(2 * DCH)]
            wr = blk * L + rl
            for f in range(F):
                vv = vs_v[wr * (FP // 8) + f // 8, pl.ds((f % 8) * L, L)]
                for c in range(DCH):
                    t = emb_v[rl, f, pl.ds(c * L, L)] * vv
                    accs[c] = accs[c] + t
                    accs[DCH + c] = accs[DCH + c] + t * t
            for c in range(DCH):
                pooled_v[row, pl.ds(c * L, L)] = accs[c]
                sumsq_v[row, pl.ds(c * L, L)] = accs[DCH + c]
            return rc

        lax.fori_loop(0, L, row_body, 0)
        return carry

    lax.fori_loop(0, NBLK, block_body, 0)

    pltpu.sync_copy(pooled_v, pooled_hbm.at[pl.ds(row0, ROWS_W)])
    pltpu.sync_copy(sumsq_v, sumsq_hbm.at[pl.ds(row0, ROWS_W)])
    pltpu.sync_copy(fow_v, fow_hbm.at[pl.ds(row0 * FP, ROWS_W * FP)])


@functools.cache
def _get_sc_call():
    return pl.kernel(
        _sc_body,
        out_type=(
            jax.ShapeDtypeStruct((B, D), jnp.float32),
            jax.ShapeDtypeStruct((B, D), jnp.float32),
            jax.ShapeDtypeStruct((B * FP,), jnp.float32),
        ),
        mesh=plsc.VectorSubcoreMesh(core_axis_name="c", subcore_axis_name="s"),
        compiler_params=pltpu.CompilerParams(use_tc_tiling_on_sc=False),
        scratch_types=(
            pltpu.VMEM((ROWS_W, F), jnp.int32),
            pltpu.VMEM((ROWS_W * FP // 8, 8 * L), jnp.float32),
            pltpu.VMEM((L, F, D), jnp.float32),
            pltpu.VMEM((ROWS_W * FP,), jnp.float32),
            pltpu.VMEM((ROWS_W, D), jnp.float32),
            pltpu.VMEM((ROWS_W, D), jnp.float32),
            pltpu.SemaphoreType.DMA,
            pltpu.SemaphoreType.DMA,
            pltpu.SemaphoreType.DMA,
        ),
    )


def _tc_body(pooled_ref, sumsq_ref, fow_ref, fv_ref,
             W0_ref, b0_ref, g0_ref, be0_ref,
             W1_ref, b1_ref, g1_ref, be1_ref,
             W2_ref, b2_ref, g2_ref, be2_ref,
             Wo_ref, bo_ref, out_ref):
    p = pooled_ref[:]
    second = 0.5 * jnp.sum(p * p - sumsq_ref[:], axis=1)
    first = jnp.sum(fow_ref[:, :F] * fv_ref[:], axis=1)
    x = p
    for W_ref, b_ref, g_ref, be_ref in (
            (W0_ref, b0_ref, g0_ref, be0_ref),
            (W1_ref, b1_ref, g1_ref, be1_ref),
            (W2_ref, b2_ref, g2_ref, be2_ref)):
        x = lax.dot_general(x, W_ref[:], (((1,), (1,)), ((), ())),
                            preferred_element_type=jnp.float32) + b_ref[:]
        x = jnp.maximum(x, 0.0)
        mean = jnp.mean(x, axis=0, keepdims=True)
        var = jnp.mean((x - mean) ** 2, axis=0, keepdims=True)
        x = g_ref[:] * (x - mean) * lax.rsqrt(var + 1e-5) + be_ref[:]
    deep = lax.dot_general(x, Wo_ref[:], (((1,), (1,)), ((), ())),
                           preferred_element_type=jnp.float32)[:, 0]
    logit = first + second + deep + bo_ref[0]
    out_ref[:] = 1.0 / (1.0 + jnp.exp(-logit))


def kernel(feature_ids, feature_values, first_order_table, second_order_table,
           W0, b0, gamma0, beta0, W1, b1, gamma1, beta1, W2, b2, gamma2, beta2,
           W_out, b_out):
    fo_flat = first_order_table.reshape(V)
    fvp = jnp.pad(feature_values, ((0, 0), (0, FP - F)))
    e_mat = jnp.repeat(jnp.eye(8, dtype=jnp.float32), L, axis=1)
    vs = lax.dot_general(fvp.reshape(B * FP // 8, 8), e_mat,
                         (((1,), (0,)), ((), ())),
                         preferred_element_type=jnp.float32)

    pooled, sumsq, fow = _get_sc_call()(feature_ids, vs, fo_flat,
                                        second_order_table)

    return pl.pallas_call(
        _tc_body,
        out_shape=jax.ShapeDtypeStruct((B,), jnp.float32),
    )(pooled, sumsq, fow.reshape(B, FP), feature_values,
      W0, b0, gamma0, beta0, W1, b1, gamma1, beta1,
      W2, b2, gamma2, beta2, W_out, b_out)


# speedup vs baseline: 2.1837x; 1.0057x over previous
import functools

import jax
import jax.numpy as jnp
from jax import lax
from jax.experimental import pallas as pl
from jax.experimental.pallas import tpu as pltpu
from jax.experimental.pallas import tpu_sc as plsc

B, F = 4096, 26
V, D = 100000, 64
FP = 32
L = 16
NC, NS = 2, 16
NW = NC * NS
ROWS_W = B // NW
NBLK = ROWS_W // L
DCH = D // L


def _sc_body(ids_hbm, vs_hbm, fo_hbm, so_hbm,
             pooled_hbm, sumsq_hbm, fow_hbm,
             ids_v, vs_v, emb_v, fow_v, pooled_v, sumsq_v,
             sem_e, sem_f, sem_v):
    wid = lax.axis_index("s") * NC + lax.axis_index("c")
    row0 = wid * ROWS_W
    pltpu.sync_copy(ids_hbm.at[pl.ds(row0, ROWS_W)], ids_v)
    pltpu.sync_copy(vs_hbm.at[pl.ds(row0, ROWS_W)], vs_v)

    def block_body(blk, carry):
        for rl in range(L):
            wr = blk * L + rl
            pltpu.async_copy(so_hbm.at[ids_v.at[wr]], emb_v.at[rl], sem_e)
        for rl in range(L):
            wr = blk * L + rl
            pltpu.async_copy(fo_hbm.at[ids_v.at[wr]],
                             fow_v.at[pl.ds(wr * FP, F)], sem_f)
        for rl in range(L):
            wr = blk * L + rl
            pltpu.make_async_copy(so_hbm.at[ids_v.at[wr]], emb_v.at[rl],
                                  sem_e).wait()
        for rl in range(L):
            wr = blk * L + rl
            pltpu.make_async_copy(fo_hbm.at[ids_v.at[wr]],
                                  fow_v.at[pl.ds(wr * FP, F)], sem_f).wait()
        def row_body(rl, rc):
            row = blk * L + rl
            accs = [jnp.zeros((L,), jnp.float32) for _ in range(2 * DCH)]
            wr = blk * L + rl
            for f in range(F):
                vv = vs_v[wr, pl.ds(f * L, L)]
                for c in range(DCH):
                    t = emb_v[rl, f, pl.ds(c * L, L)] * vv
                    accs[c] = accs[c] + t
                    accs[DCH + c] = accs[DCH + c] + t * t
            for c in range(DCH):
                pooled_v[row, pl.ds(c * L, L)] = accs[c]
                sumsq_v[row, pl.ds(c * L, L)] = accs[DCH + c]
            return rc

        lax.fori_loop(0, L, row_body, 0)
        return carry

    lax.fori_loop(0, NBLK, block_body, 0)

    pltpu.sync_copy(pooled_v, pooled_hbm.at[pl.ds(row0, ROWS_W)])
    pltpu.sync_copy(sumsq_v, sumsq_hbm.at[pl.ds(row0, ROWS_W)])
    pltpu.sync_copy(fow_v, fow_hbm.at[pl.ds(row0 * FP, ROWS_W * FP)])


@functools.cache
def _get_sc_call():
    return pl.kernel(
        _sc_body,
        out_type=(
            jax.ShapeDtypeStruct((B, D), jnp.float32),
            jax.ShapeDtypeStruct((B, D), jnp.float32),
            jax.ShapeDtypeStruct((B * FP,), jnp.float32),
        ),
        mesh=plsc.VectorSubcoreMesh(core_axis_name="c", subcore_axis_name="s"),
        compiler_params=pltpu.CompilerParams(use_tc_tiling_on_sc=False),
        scratch_types=(
            pltpu.VMEM((ROWS_W, F), jnp.int32),
            pltpu.VMEM((ROWS_W, FP * L), jnp.float32),
            pltpu.VMEM((L, F, D), jnp.float32),
            pltpu.VMEM((ROWS_W * FP,), jnp.float32),
            pltpu.VMEM((ROWS_W, D), jnp.float32),
            pltpu.VMEM((ROWS_W, D), jnp.float32),
            pltpu.SemaphoreType.DMA,
            pltpu.SemaphoreType.DMA,
            pltpu.SemaphoreType.DMA,
        ),
    )


def _tc_body(pooled_ref, sumsq_ref, fow_ref, fv_ref,
             W0_ref, b0_ref, g0_ref, be0_ref,
             W1_ref, b1_ref, g1_ref, be1_ref,
             W2_ref, b2_ref, g2_ref, be2_ref,
             Wo_ref, bo_ref, out_ref):
    p = pooled_ref[:]
    second = 0.5 * jnp.sum(p * p - sumsq_ref[:], axis=1)
    first = jnp.sum(fow_ref[:, :F] * fv_ref[:], axis=1)
    x = p
    for W_ref, b_ref, g_ref, be_ref in (
            (W0_ref, b0_ref, g0_ref, be0_ref),
            (W1_ref, b1_ref, g1_ref, be1_ref),
            (W2_ref, b2_ref, g2_ref, be2_ref)):
        x = lax.dot_general(x, W_ref[:], (((1,), (1,)), ((), ())),
                            preferred_element_type=jnp.float32) + b_ref[:]
        x = jnp.maximum(x, 0.0)
        mean = jnp.mean(x, axis=0, keepdims=True)
        var = jnp.mean((x - mean) ** 2, axis=0, keepdims=True)
        x = g_ref[:] * (x - mean) * lax.rsqrt(var + 1e-5) + be_ref[:]
    deep = lax.dot_general(x, Wo_ref[:], (((1,), (1,)), ((), ())),
                           preferred_element_type=jnp.float32)[:, 0]
    logit = first + second + deep + bo_ref[0]
    out_ref[:] = 1.0 / (1.0 + jnp.exp(-logit))


def kernel(feature_ids, feature_values, first_order_table, second_order_table,
           W0, b0, gamma0, beta0, W1, b1, gamma1, beta1, W2, b2, gamma2, beta2,
           W_out, b_out):
    fo_flat = first_order_table.reshape(V)
    g_mat = jnp.repeat(jnp.eye(F, FP, dtype=jnp.float32), L, axis=1)
    vs = lax.dot_general(feature_values, g_mat, (((1,), (0,)), ((), ())),
                         preferred_element_type=jnp.float32)

    pooled, sumsq, fow = _get_sc_call()(feature_ids, vs, fo_flat,
                                        second_order_table)

    return pl.pallas_call(
        _tc_body,
        out_shape=jax.ShapeDtypeStruct((B,), jnp.float32),
    )(pooled, sumsq, fow.reshape(B, FP), feature_values,
      W0, b0, gamma0, beta0, W1, b1, gamma1, beta1,
      W2, b2, gamma2, beta2, W_out, b_out)


# speedup vs baseline: 2.2010x; 1.0079x over previous
import functools

import jax
import jax.numpy as jnp
from jax import lax
from jax.experimental import pallas as pl
from jax.experimental.pallas import tpu as pltpu
from jax.experimental.pallas import tpu_sc as plsc

B, F = 4096, 26
V, D = 100000, 64
FP = 32
L = 16
NC, NS = 2, 16
NW = NC * NS
ROWS_W = B // NW
NBLK = ROWS_W // L
DCH = D // L


def _sc_body(ids_hbm, vs_hbm, so_hbm,
             pooled_hbm, sumsq_hbm,
             ids_v, vs_v, emb_v, pooled_v, sumsq_v,
             sem_e):
    wid = lax.axis_index("s") * NC + lax.axis_index("c")
    row0 = wid * ROWS_W
    pltpu.sync_copy(ids_hbm.at[pl.ds(row0, ROWS_W)], ids_v)
    pltpu.sync_copy(vs_hbm.at[pl.ds(row0, ROWS_W)], vs_v)

    def block_body(blk, carry):
        for rl in range(L):
            wr = blk * L + rl
            pltpu.async_copy(so_hbm.at[ids_v.at[wr]], emb_v.at[rl], sem_e)
        for rl in range(L):
            wr = blk * L + rl
            pltpu.make_async_copy(so_hbm.at[ids_v.at[wr]], emb_v.at[rl],
                                  sem_e).wait()
        def row_body(rl, rc):
            row = blk * L + rl
            accs = [jnp.zeros((L,), jnp.float32) for _ in range(2 * DCH)]
            wr = blk * L + rl
            for f in range(F):
                vv = vs_v[wr, pl.ds(f * L, L)]
                for c in range(DCH):
                    t = emb_v[rl, f, pl.ds(c * L, L)] * vv
                    accs[c] = accs[c] + t
                    accs[DCH + c] = accs[DCH + c] + t * t
            for c in range(DCH):
                pooled_v[row, pl.ds(c * L, L)] = accs[c]
                sumsq_v[row, pl.ds(c * L, L)] = accs[DCH + c]
            return rc

        lax.fori_loop(0, L, row_body, 0)
        return carry

    lax.fori_loop(0, NBLK, block_body, 0)

    pltpu.sync_copy(pooled_v, pooled_hbm.at[pl.ds(row0, ROWS_W)])
    pltpu.sync_copy(sumsq_v, sumsq_hbm.at[pl.ds(row0, ROWS_W)])


def _sc_fo_body(ids_hbm, fo_hbm, fow_hbm, ids_v, fow_v, sem_f):
    wid = lax.axis_index("s") * NC + lax.axis_index("c")
    row0 = wid * ROWS_W
    pltpu.sync_copy(ids_hbm.at[pl.ds(row0, ROWS_W)], ids_v)
    for wr in range(ROWS_W):
        pltpu.async_copy(fo_hbm.at[ids_v.at[wr]],
                         fow_v.at[pl.ds(wr * FP, F)], sem_f)
    for wr in range(ROWS_W):
        pltpu.make_async_copy(fo_hbm.at[ids_v.at[wr]],
                              fow_v.at[pl.ds(wr * FP, F)], sem_f).wait()
    pltpu.sync_copy(fow_v, fow_hbm.at[pl.ds(row0 * FP, ROWS_W * FP)])


@functools.cache
def _get_sc_call():
    return pl.kernel(
        _sc_body,
        out_type=(
            jax.ShapeDtypeStruct((B, D), jnp.float32),
            jax.ShapeDtypeStruct((B, D), jnp.float32),
        ),
        mesh=plsc.VectorSubcoreMesh(core_axis_name="c", subcore_axis_name="s"),
        compiler_params=pltpu.CompilerParams(use_tc_tiling_on_sc=False),
        scratch_types=(
            pltpu.VMEM((ROWS_W, F), jnp.int32),
            pltpu.VMEM((ROWS_W, FP * L), jnp.float32),
            pltpu.VMEM((L, F, D), jnp.float32),
            pltpu.VMEM((ROWS_W, D), jnp.float32),
            pltpu.VMEM((ROWS_W, D), jnp.float32),
            pltpu.SemaphoreType.DMA,
        ),
    )


@functools.cache
def _get_sc_fo_call():
    return pl.kernel(
        _sc_fo_body,
        out_type=(jax.ShapeDtypeStruct((B * FP,), jnp.float32),),
        mesh=plsc.VectorSubcoreMesh(core_axis_name="c", subcore_axis_name="s"),
        compiler_params=pltpu.CompilerParams(use_tc_tiling_on_sc=False),
        scratch_types=(
            pltpu.VMEM((ROWS_W, F), jnp.int32),
            pltpu.VMEM((ROWS_W * FP,), jnp.float32),
            pltpu.SemaphoreType.DMA,
        ),
    )


def _tc_body(pooled_ref, sumsq_ref, fow_ref, fv_ref,
             W0_ref, b0_ref, g0_ref, be0_ref,
             W1_ref, b1_ref, g1_ref, be1_ref,
             W2_ref, b2_ref, g2_ref, be2_ref,
             Wo_ref, bo_ref, out_ref):
    p = pooled_ref[:]
    second = 0.5 * jnp.sum(p * p - sumsq_ref[:], axis=1)
    first = jnp.sum(fow_ref[:, :F] * fv_ref[:], axis=1)
    x = p
    for W_ref, b_ref, g_ref, be_ref in (
            (W0_ref, b0_ref, g0_ref, be0_ref),
            (W1_ref, b1_ref, g1_ref, be1_ref),
            (W2_ref, b2_ref, g2_ref, be2_ref)):
        x = lax.dot_general(x, W_ref[:], (((1,), (1,)), ((), ())),
                            preferred_element_type=jnp.float32) + b_ref[:]
        x = jnp.maximum(x, 0.0)
        mean = jnp.mean(x, axis=0, keepdims=True)
        var = jnp.mean((x - mean) ** 2, axis=0, keepdims=True)
        x = g_ref[:] * (x - mean) * lax.rsqrt(var + 1e-5) + be_ref[:]
    deep = lax.dot_general(x, Wo_ref[:], (((1,), (1,)), ((), ())),
                           preferred_element_type=jnp.float32)[:, 0]
    logit = first + second + deep + bo_ref[0]
    out_ref[:] = 1.0 / (1.0 + jnp.exp(-logit))


def kernel(feature_ids, feature_values, first_order_table, second_order_table,
           W0, b0, gamma0, beta0, W1, b1, gamma1, beta1, W2, b2, gamma2, beta2,
           W_out, b_out):
    g_mat = jnp.repeat(jnp.eye(F, FP, dtype=jnp.float32), L, axis=1)
    vs = lax.dot_general(feature_values, g_mat, (((1,), (0,)), ((), ())),
                         preferred_element_type=jnp.float32)

    pooled, sumsq = _get_sc_call()(feature_ids, vs, second_order_table)
    (fow,) = _get_sc_fo_call()(feature_ids, first_order_table.reshape(V))

    return pl.pallas_call(
        _tc_body,
        out_shape=jax.ShapeDtypeStruct((B,), jnp.float32),
    )(pooled, sumsq, fow.reshape(B, FP), feature_values,
      W0, b0, gamma0, beta0, W1, b1, gamma1, beta1,
      W2, b2, gamma2, beta2, W_out, b_out)


# speedup vs baseline: 2.2133x; 1.0056x over previous
import functools

import jax
import jax.numpy as jnp
from jax import lax
from jax.experimental import pallas as pl
from jax.experimental.pallas import tpu as pltpu
from jax.experimental.pallas import tpu_sc as plsc

B, F = 4096, 26
V, D = 100000, 64
FP = 32
NCH = 4
CH = L * 26 // NCH if False else 104
CHF = 128
L = 16
NC, NS = 2, 16
NW = NC * NS
ROWS_W = B // NW
NBLK = ROWS_W // L
DCH = D // L


def _sc_body(ids_hbm, vs_hbm, so_hbm,
             pooled_hbm, sumsq_hbm,
             ids_v, vs_v, emb_v, pooled_v, sumsq_v,
             sem_e):
    wid = lax.axis_index("s") * NC + lax.axis_index("c")
    row0 = wid * ROWS_W
    pltpu.sync_copy(ids_hbm.at[pl.ds(row0 * F, ROWS_W * F)], ids_v)
    pltpu.sync_copy(vs_hbm.at[pl.ds(row0, ROWS_W)], vs_v)

    def block_body(blk, carry):
        for c in range(NCH):
            pltpu.async_copy(
                so_hbm.at[ids_v.at[pl.ds(blk * (L * F) + c * CH, CH)]],
                emb_v.at[pl.ds(c * CH, CH)], sem_e)
        for c in range(NCH):
            pltpu.make_async_copy(
                so_hbm.at[ids_v.at[pl.ds(blk * (L * F) + c * CH, CH)]],
                emb_v.at[pl.ds(c * CH, CH)], sem_e).wait()
        def row_body(rl, rc):
            row = blk * L + rl
            accs = [jnp.zeros((L,), jnp.float32) for _ in range(2 * DCH)]
            wr = blk * L + rl
            for f in range(F):
                vv = vs_v[wr, pl.ds(f * L, L)]
                for c in range(DCH):
                    t = emb_v[rl * F + f, pl.ds(c * L, L)] * vv
                    accs[c] = accs[c] + t
                    accs[DCH + c] = accs[DCH + c] + t * t
            for c in range(DCH):
                pooled_v[row, pl.ds(c * L, L)] = accs[c]
                sumsq_v[row, pl.ds(c * L, L)] = accs[DCH + c]
            return rc

        lax.fori_loop(0, L, row_body, 0)
        return carry

    lax.fori_loop(0, NBLK, block_body, 0)

    pltpu.sync_copy(pooled_v, pooled_hbm.at[pl.ds(row0, ROWS_W)])
    pltpu.sync_copy(sumsq_v, sumsq_hbm.at[pl.ds(row0, ROWS_W)])


def _sc_fo_body(ids_hbm, fo_hbm, fow_hbm, ids_v, fow_v, sem_f):
    wid = lax.axis_index("s") * NC + lax.axis_index("c")
    base = wid * (ROWS_W * F)
    pltpu.sync_copy(ids_hbm.at[pl.ds(base, ROWS_W * F)], ids_v)
    for k in range(ROWS_W * F // CHF):
        pltpu.async_copy(fo_hbm.at[ids_v.at[pl.ds(k * CHF, CHF)]],
                         fow_v.at[pl.ds(k * CHF, CHF)], sem_f)
    for k in range(ROWS_W * F // CHF):
        pltpu.make_async_copy(fo_hbm.at[ids_v.at[pl.ds(k * CHF, CHF)]],
                              fow_v.at[pl.ds(k * CHF, CHF)], sem_f).wait()
    pltpu.sync_copy(fow_v, fow_hbm.at[pl.ds(base, ROWS_W * F)])


@functools.cache
def _get_sc_call():
    return pl.kernel(
        _sc_body,
        out_type=(
            jax.ShapeDtypeStruct((B, D), jnp.float32),
            jax.ShapeDtypeStruct((B, D), jnp.float32),
        ),
        mesh=plsc.VectorSubcoreMesh(core_axis_name="c", subcore_axis_name="s"),
        compiler_params=pltpu.CompilerParams(use_tc_tiling_on_sc=False),
        scratch_types=(
            pltpu.VMEM((ROWS_W * F,), jnp.int32),
            pltpu.VMEM((ROWS_W, FP * L), jnp.float32),
            pltpu.VMEM((L * F, D), jnp.float32),
            pltpu.VMEM((ROWS_W, D), jnp.float32),
            pltpu.VMEM((ROWS_W, D), jnp.float32),
            pltpu.SemaphoreType.DMA,
        ),
    )


@functools.cache
def _get_sc_fo_call():
    return pl.kernel(
        _sc_fo_body,
        out_type=(jax.ShapeDtypeStruct((B * F,), jnp.float32),),
        mesh=plsc.VectorSubcoreMesh(core_axis_name="c", subcore_axis_name="s"),
        compiler_params=pltpu.CompilerParams(use_tc_tiling_on_sc=False),
        scratch_types=(
            pltpu.VMEM((ROWS_W * F,), jnp.int32),
            pltpu.VMEM((ROWS_W * F,), jnp.float32),
            pltpu.SemaphoreType.DMA,
        ),
    )


def _tc_body(pooled_ref, sumsq_ref, fow_ref, fv_ref,
             W0_ref, b0_ref, g0_ref, be0_ref,
             W1_ref, b1_ref, g1_ref, be1_ref,
             W2_ref, b2_ref, g2_ref, be2_ref,
             Wo_ref, bo_ref, out_ref):
    p = pooled_ref[:]
    second = 0.5 * jnp.sum(p * p - sumsq_ref[:], axis=1)
    first = jnp.sum(fow_ref[:] * fv_ref[:], axis=1)
    x = p
    for W_ref, b_ref, g_ref, be_ref in (
            (W0_ref, b0_ref, g0_ref, be0_ref),
            (W1_ref, b1_ref, g1_ref, be1_ref),
            (W2_ref, b2_ref, g2_ref, be2_ref)):
        x = lax.dot_general(x, W_ref[:], (((1,), (1,)), ((), ())),
                            preferred_element_type=jnp.float32) + b_ref[:]
        x = jnp.maximum(x, 0.0)
        mean = jnp.mean(x, axis=0, keepdims=True)
        var = jnp.mean((x - mean) ** 2, axis=0, keepdims=True)
        x = g_ref[:] * (x - mean) * lax.rsqrt(var + 1e-5) + be_ref[:]
    deep = lax.dot_general(x, Wo_ref[:], (((1,), (1,)), ((), ())),
                           preferred_element_type=jnp.float32)[:, 0]
    logit = first + second + deep + bo_ref[0]
    out_ref[:] = 1.0 / (1.0 + jnp.exp(-logit))


def kernel(feature_ids, feature_values, first_order_table, second_order_table,
           W0, b0, gamma0, beta0, W1, b1, gamma1, beta1, W2, b2, gamma2, beta2,
           W_out, b_out):
    g_mat = jnp.repeat(jnp.eye(F, FP, dtype=jnp.float32), L, axis=1)
    vs = lax.dot_general(feature_values, g_mat, (((1,), (0,)), ((), ())),
                         preferred_element_type=jnp.float32)

    ids1d = feature_ids.reshape(B * F)
    pooled, sumsq = _get_sc_call()(ids1d, vs, second_order_table)
    (fow,) = _get_sc_fo_call()(ids1d, first_order_table.reshape(V))

    return pl.pallas_call(
        _tc_body,
        out_shape=jax.ShapeDtypeStruct((B,), jnp.float32),
    )(pooled, sumsq, fow.reshape(B, F), feature_values,
      W0, b0, gamma0, beta0, W1, b1, gamma1, beta1,
      W2, b2, gamma2, beta2, W_out, b_out)
